# baseline (speedup 1.0000x reference)
import jax
import jax.numpy as jnp
from jax.experimental import pallas as pl
from jax.experimental.pallas import tpu as pltpu

_B = 16384
_IN_DIM = 384
_HIDDEN = 512
_LATENT = 64
_CB_SIZE = 1024
_NUM_Q = 4
_BLK = 1024


def _mm(a, b):
    return jnp.dot(a.astype(jnp.bfloat16), b.astype(jnp.bfloat16),
                   preferred_element_type=jnp.float32)


def _rqvae_block(x_ref, ew1, eb1, ew2, eb2, ew3, eb3,
                 dw1, db1, dw2, db2, dw3, db3, cb_ref,
                 xr_ref, idx_ref, loss_ref):
    i = pl.program_id(0)

    x = x_ref[...]
    h = jnp.maximum(_mm(x, ew1[...]) + eb1[...], 0.0)
    h = jnp.maximum(_mm(h, ew2[...]) + eb2[...], 0.0)
    z = _mm(h, ew3[...]) + eb3[...]

    residual = z
    quant = jnp.zeros_like(z)
    losses = []
    for q in range(_NUM_Q):
        cb = cb_ref[q]
        r_sq = jnp.sum(residual * residual, axis=-1, keepdims=True)
        cb_sq = jnp.sum(cb * cb, axis=-1)[None, :]
        d = r_sq - 2.0 * _mm(residual, cb.T) + cb_sq
        idx = jnp.argmin(d, axis=-1).astype(jnp.int32)
        onehot = (jax.lax.broadcasted_iota(jnp.int32, (_BLK, _CB_SIZE), 1)
                  == idx[:, None]).astype(jnp.float32)
        qv = jnp.dot(onehot, cb, preferred_element_type=jnp.float32,
                     precision=jax.lax.Precision.HIGHEST)
        diff = qv - residual
        losses.append(jnp.sum(diff * diff))
        quant = quant + qv
        residual = residual - qv
        idx_ref[q, :] = idx

    loss_vec = jnp.stack(losses)

    @pl.when(i == 0)
    def _():
        loss_ref[...] = jnp.zeros_like(loss_ref)

    loss_ref[...] += loss_vec[:, None]

    h = jnp.maximum(_mm(quant, dw1[...]) + db1[...], 0.0)
    h = jnp.maximum(_mm(h, dw2[...]) + db2[...], 0.0)
    xr_ref[...] = _mm(h, dw3[...]) + db3[...]


def kernel(x, enc_w1, enc_b1, enc_w2, enc_b2, enc_w3, enc_b3,
           dec_w1, dec_b1, dec_w2, dec_b2, dec_w3, dec_b3, codebooks):
    grid = _B // _BLK

    def _full(shape):
        nd = len(shape)
        return pl.BlockSpec(shape, lambda i, _nd=nd: (0,) * _nd)

    ew1t = enc_w1.T
    ew2t = enc_w2.T
    ew3t = enc_w3.T
    dw1t = dec_w1.T
    dw2t = dec_w2.T
    dw3t = dec_w3.T
    eb1r = enc_b1.reshape(1, -1)
    eb2r = enc_b2.reshape(1, -1)
    eb3r = enc_b3.reshape(1, -1)
    db1r = dec_b1.reshape(1, -1)
    db2r = dec_b2.reshape(1, -1)
    db3r = dec_b3.reshape(1, -1)

    x_recon, idx_out, loss_out = pl.pallas_call(
        _rqvae_block,
        grid=(grid,),
        in_specs=[
            pl.BlockSpec((_BLK, _IN_DIM), lambda i: (i, 0)),
            _full(ew1t.shape), _full(eb1r.shape),
            _full(ew2t.shape), _full(eb2r.shape),
            _full(ew3t.shape), _full(eb3r.shape),
            _full(dw1t.shape), _full(db1r.shape),
            _full(dw2t.shape), _full(db2r.shape),
            _full(dw3t.shape), _full(db3r.shape),
            _full(codebooks.shape),
        ],
        out_specs=[
            pl.BlockSpec((_BLK, _IN_DIM), lambda i: (i, 0)),
            pl.BlockSpec((_NUM_Q, _BLK), lambda i: (0, i)),
            pl.BlockSpec((_NUM_Q, 128), lambda i: (0, 0)),
        ],
        out_shape=[
            jax.ShapeDtypeStruct((_B, _IN_DIM), jnp.float32),
            jax.ShapeDtypeStruct((_NUM_Q, _B), jnp.int32),
            jax.ShapeDtypeStruct((_NUM_Q, 128), jnp.float32),
        ],
    )(x, ew1t, eb1r, ew2t, eb2r, ew3t, eb3r,
      dw1t, db1r, dw2t, db2r, dw3t, db3r, codebooks)

    indices = idx_out.T
    commitment_loss = loss_out[:, 0] / (_B * _LATENT)
    return x_recon, indices, commitment_loss


# speedup vs baseline: 2.0795x; 2.0795x over previous
import jax
import jax.numpy as jnp
from jax.experimental import pallas as pl
from jax.experimental.pallas import tpu as pltpu

_B = 16384
_IN_DIM = 384
_HIDDEN = 512
_LATENT = 64
_CB_SIZE = 1024
_NUM_Q = 4
_BLK = 1024


def _mm(a, b):
    return jnp.dot(a.astype(jnp.bfloat16), b.astype(jnp.bfloat16),
                   preferred_element_type=jnp.float32)


def _rqvae_block(x_ref, ew1, eb1, ew2, eb2, ew3, eb3,
                 dw1, db1, dw2, db2, dw3, db3, cb_ref, cbt_ref,
                 cb12_ref, cb3_ref,
                 xr_ref, idx_ref, loss_ref):
    i = pl.program_id(0)

    x = x_ref[...]
    h = jnp.maximum(_mm(x, ew1[...]) + eb1[...], 0.0)
    h = jnp.maximum(_mm(h, ew2[...]) + eb2[...], 0.0)
    z = _mm(h, ew3[...]) + eb3[...]

    residual = z
    quant = jnp.zeros_like(z)
    losses = []
    for q in range(_NUM_Q):
        cb = cb_ref[q]
        r_sq = jnp.sum(residual * residual, axis=-1, keepdims=True)
        cb_sq = jnp.sum(cb * cb, axis=-1)[None, :]
        d = r_sq - 2.0 * _mm(residual, cbt_ref[q]) + cb_sq
        idx = jnp.argmin(d, axis=-1).astype(jnp.int32)
        onehot = (jax.lax.broadcasted_iota(jnp.int32, (_BLK, _CB_SIZE), 1)
                  == idx[:, None]).astype(jnp.bfloat16)
        gA = jnp.dot(onehot, cb12_ref[q], preferred_element_type=jnp.float32)
        gB = jnp.dot(onehot, cb3_ref[q], preferred_element_type=jnp.float32)
        qv = (gA[:, :_LATENT] + gA[:, _LATENT:]) + gB
        diff = qv - residual
        losses.append(jnp.sum(diff * diff))
        quant = quant + qv
        residual = residual - qv
        idx_ref[q, :] = idx

    loss_vec = jnp.stack(losses)

    @pl.when(i == 0)
    def _():
        loss_ref[...] = jnp.zeros_like(loss_ref)

    loss_ref[...] += loss_vec[:, None]

    h = jnp.maximum(_mm(quant, dw1[...]) + db1[...], 0.0)
    h = jnp.maximum(_mm(h, dw2[...]) + db2[...], 0.0)
    xr_ref[...] = _mm(h, dw3[...]) + db3[...]


def kernel(x, enc_w1, enc_b1, enc_w2, enc_b2, enc_w3, enc_b3,
           dec_w1, dec_b1, dec_w2, dec_b2, dec_w3, dec_b3, codebooks):
    grid = _B // _BLK

    def _full(shape):
        nd = len(shape)
        return pl.BlockSpec(shape, lambda i, _nd=nd: (0,) * _nd)

    ew1t = enc_w1.T
    ew2t = enc_w2.T
    ew3t = enc_w3.T
    dw1t = dec_w1.T
    dw2t = dec_w2.T
    dw3t = dec_w3.T
    eb1r = enc_b1.reshape(1, -1)
    eb2r = enc_b2.reshape(1, -1)
    eb3r = enc_b3.reshape(1, -1)
    db1r = dec_b1.reshape(1, -1)
    db2r = dec_b2.reshape(1, -1)
    db3r = dec_b3.reshape(1, -1)

    c1 = codebooks.astype(jnp.bfloat16)
    c2 = (codebooks - c1.astype(jnp.float32)).astype(jnp.bfloat16)
    c3 = (codebooks - c1.astype(jnp.float32) - c2.astype(jnp.float32)
          ).astype(jnp.bfloat16)
    cb12 = jnp.concatenate([c1, c2], axis=-1)
    cbt = jnp.swapaxes(codebooks, 1, 2)

    x_recon, idx_out, loss_out = pl.pallas_call(
        _rqvae_block,
        grid=(grid,),
        in_specs=[
            pl.BlockSpec((_BLK, _IN_DIM), lambda i: (i, 0)),
            _full(ew1t.shape), _full(eb1r.shape),
            _full(ew2t.shape), _full(eb2r.shape),
            _full(ew3t.shape), _full(eb3r.shape),
            _full(dw1t.shape), _full(db1r.shape),
            _full(dw2t.shape), _full(db2r.shape),
            _full(dw3t.shape), _full(db3r.shape),
            _full(codebooks.shape),
            _full(cbt.shape),
            _full(cb12.shape),
            _full(c3.shape),
        ],
        out_specs=[
            pl.BlockSpec((_BLK, _IN_DIM), lambda i: (i, 0)),
            pl.BlockSpec((_NUM_Q, _BLK), lambda i: (0, i)),
            pl.BlockSpec((_NUM_Q, 128), lambda i: (0, 0)),
        ],
        out_shape=[
            jax.ShapeDtypeStruct((_B, _IN_DIM), jnp.float32),
            jax.ShapeDtypeStruct((_NUM_Q, _B), jnp.int32),
            jax.ShapeDtypeStruct((_NUM_Q, 128), jnp.float32),
        ],
    )(x, ew1t, eb1r, ew2t, eb2r, ew3t, eb3r,
      dw1t, db1r, dw2t, db2r, dw3t, db3r, codebooks, cbt, cb12, c3)

    indices = idx_out.T
    commitment_loss = loss_out[:, 0] / (_B * _LATENT)
    return x_recon, indices, commitment_loss


# speedup vs baseline: 2.4832x; 1.1941x over previous
import jax
import jax.numpy as jnp
from jax.experimental import pallas as pl
from jax.experimental.pallas import tpu as pltpu

_B = 16384
_IN_DIM = 384
_HIDDEN = 512
_LATENT = 64
_CB_SIZE = 1024
_NUM_Q = 4
_BLK = 2048


def _mm(a, b):
    return jnp.dot(a.astype(jnp.bfloat16), b.astype(jnp.bfloat16),
                   preferred_element_type=jnp.float32)


def _rqvae_block(x_ref, ew1, eb1, ew2, eb2, ew3, eb3,
                 dw1, db1, dw2, db2, dw3, db3, cb_ref, cbt_ref,
                 cb12_ref, cb3_ref,
                 xr_ref, idx_ref, loss_ref):
    i = pl.program_id(0)

    x = x_ref[...]
    h = jnp.maximum(_mm(x, ew1[...]) + eb1[...], 0.0)
    h = jnp.maximum(_mm(h, ew2[...]) + eb2[...], 0.0)
    z = _mm(h, ew3[...]) + eb3[...]

    residual = z
    quant = jnp.zeros_like(z)
    losses = []
    for q in range(_NUM_Q):
        cb = cb_ref[q]
        r_sq = jnp.sum(residual * residual, axis=-1, keepdims=True)
        cb_sq = jnp.sum(cb * cb, axis=-1)[None, :]
        d = r_sq - 2.0 * _mm(residual, cbt_ref[q]) + cb_sq
        idx = jnp.argmin(d, axis=-1).astype(jnp.int32)
        onehot = (jax.lax.broadcasted_iota(jnp.int32, (_BLK, _CB_SIZE), 1)
                  == idx[:, None]).astype(jnp.bfloat16)
        gA = jnp.dot(onehot, cb12_ref[q], preferred_element_type=jnp.float32)
        gB = jnp.dot(onehot, cb3_ref[q], preferred_element_type=jnp.float32)
        qv = (gA[:, :_LATENT] + gA[:, _LATENT:]) + gB
        diff = qv - residual
        losses.append(jnp.sum(diff * diff))
        quant = quant + qv
        residual = residual - qv
        idx_ref[q, :] = idx

    loss_vec = jnp.stack(losses)

    @pl.when(i == 0)
    def _():
        loss_ref[...] = jnp.zeros_like(loss_ref)

    loss_ref[...] += loss_vec[:, None]

    h = jnp.maximum(_mm(quant, dw1[...]) + db1[...], 0.0)
    h = jnp.maximum(_mm(h, dw2[...]) + db2[...], 0.0)
    xr_ref[...] = _mm(h, dw3[...]) + db3[...]


def kernel(x, enc_w1, enc_b1, enc_w2, enc_b2, enc_w3, enc_b3,
           dec_w1, dec_b1, dec_w2, dec_b2, dec_w3, dec_b3, codebooks):
    grid = _B // _BLK

    def _full(shape):
        nd = len(shape)
        return pl.BlockSpec(shape, lambda i, _nd=nd: (0,) * _nd)

    ew1t = enc_w1.T
    ew2t = enc_w2.T
    ew3t = enc_w3.T
    dw1t = dec_w1.T
    dw2t = dec_w2.T
    dw3t = dec_w3.T
    eb1r = enc_b1.reshape(1, -1)
    eb2r = enc_b2.reshape(1, -1)
    eb3r = enc_b3.reshape(1, -1)
    db1r = dec_b1.reshape(1, -1)
    db2r = dec_b2.reshape(1, -1)
    db3r = dec_b3.reshape(1, -1)

    c1 = codebooks.astype(jnp.bfloat16)
    c2 = (codebooks - c1.astype(jnp.float32)).astype(jnp.bfloat16)
    c3 = (codebooks - c1.astype(jnp.float32) - c2.astype(jnp.float32)
          ).astype(jnp.bfloat16)
    cb12 = jnp.concatenate([c1, c2], axis=-1)
    cbt = jnp.swapaxes(codebooks, 1, 2)

    x_recon, idx_out, loss_out = pl.pallas_call(
        _rqvae_block,
        grid=(grid,),
        in_specs=[
            pl.BlockSpec((_BLK, _IN_DIM), lambda i: (i, 0)),
            _full(ew1t.shape), _full(eb1r.shape),
            _full(ew2t.shape), _full(eb2r.shape),
            _full(ew3t.shape), _full(eb3r.shape),
            _full(dw1t.shape), _full(db1r.shape),
            _full(dw2t.shape), _full(db2r.shape),
            _full(dw3t.shape), _full(db3r.shape),
            _full(codebooks.shape),
            _full(cbt.shape),
            _full(cb12.shape),
            _full(c3.shape),
        ],
        out_specs=[
            pl.BlockSpec((_BLK, _IN_DIM), lambda i: (i, 0)),
            pl.BlockSpec((_NUM_Q, _BLK), lambda i: (0, i)),
            pl.BlockSpec((_NUM_Q, 128), lambda i: (0, 0)),
        ],
        out_shape=[
            jax.ShapeDtypeStruct((_B, _IN_DIM), jnp.float32),
            jax.ShapeDtypeStruct((_NUM_Q, _B), jnp.int32),
            jax.ShapeDtypeStruct((_NUM_Q, 128), jnp.float32),
        ],
    )(x, ew1t, eb1r, ew2t, eb2r, ew3t, eb3r,
      dw1t, db1r, dw2t, db2r, dw3t, db3r, codebooks, cbt, cb12, c3)

    indices = idx_out.T
    commitment_loss = loss_out[:, 0] / (_B * _LATENT)
    return x_recon, indices, commitment_loss


# speedup vs baseline: 2.4884x; 1.0021x over previous
import jax
import jax.numpy as jnp
from jax.experimental import pallas as pl
from jax.experimental.pallas import tpu as pltpu

_B = 16384
_IN_DIM = 384
_HIDDEN = 512
_LATENT = 64
_CB_SIZE = 1024
_NUM_Q = 4
_BLK = 2048


def _mm(a, b):
    return jnp.dot(a.astype(jnp.bfloat16), b.astype(jnp.bfloat16),
                   preferred_element_type=jnp.float32)


def _rqvae_block(x_ref, ew1, eb1, ew2, eb2, ew3, eb3,
                 dw1, db1, dw2, db2, dw3, db3, cb_ref, cbt_ref,
                 cb12_ref, cb3_ref,
                 xr_ref, idx_ref, loss_ref):
    i = pl.program_id(0)

    x = x_ref[...]
    h = jnp.maximum(_mm(x, ew1[...]) + eb1[...], 0.0)
    h = jnp.maximum(_mm(h, ew2[...]) + eb2[...], 0.0)
    z = _mm(h, ew3[...]) + eb3[...]

    residual = z
    quant = jnp.zeros_like(z)
    losses = []
    for q in range(_NUM_Q):
        cb = cb_ref[q]
        r_sq = jnp.sum(residual * residual, axis=-1, keepdims=True)
        cb_sq = jnp.sum(cb * cb, axis=-1)[None, :]
        d = r_sq - 2.0 * _mm(residual, cbt_ref[q]) + cb_sq
        idx = jnp.argmin(d, axis=-1).astype(jnp.int32)
        onehot = (jax.lax.broadcasted_iota(jnp.int32, (_BLK, _CB_SIZE), 1)
                  == idx[:, None]).astype(jnp.bfloat16)
        gA = jnp.dot(onehot, cb12_ref[q], preferred_element_type=jnp.float32)
        gB = jnp.dot(onehot, cb3_ref[q], preferred_element_type=jnp.float32)
        qv = (gA[:, :_LATENT] + gA[:, _LATENT:]) + gB
        diff = qv - residual
        losses.append(jnp.sum(diff * diff))
        quant = quant + qv
        residual = residual - qv
        idx_ref[q, :] = idx

    loss_vec = jnp.stack(losses)

    @pl.when(i == 0)
    def _():
        loss_ref[...] = jnp.zeros_like(loss_ref)

    loss_ref[...] += loss_vec[:, None]

    h = jnp.maximum(_mm(quant, dw1[...]) + db1[...], 0.0)
    h = jnp.maximum(_mm(h, dw2[...]) + db2[...], 0.0)
    xr_ref[...] = _mm(h, dw3[...]) + db3[...]


def kernel(x, enc_w1, enc_b1, enc_w2, enc_b2, enc_w3, enc_b3,
           dec_w1, dec_b1, dec_w2, dec_b2, dec_w3, dec_b3, codebooks):
    grid = _B // _BLK

    def _full(shape):
        nd = len(shape)
        return pl.BlockSpec(shape, lambda i, _nd=nd: (0,) * _nd)

    ew1t = enc_w1.T
    ew2t = enc_w2.T
    ew3t = enc_w3.T
    dw1t = dec_w1.T
    dw2t = dec_w2.T
    dw3t = dec_w3.T
    eb1r = enc_b1.reshape(1, -1)
    eb2r = enc_b2.reshape(1, -1)
    eb3r = enc_b3.reshape(1, -1)
    db1r = dec_b1.reshape(1, -1)
    db2r = dec_b2.reshape(1, -1)
    db3r = dec_b3.reshape(1, -1)

    mask = jnp.uint32(0xFFFF0000)
    b0 = jax.lax.bitcast_convert_type(codebooks, jnp.uint32)
    c1f = jax.lax.bitcast_convert_type(b0 & mask, jnp.float32)
    r1 = codebooks - c1f
    c2f = jax.lax.bitcast_convert_type(
        jax.lax.bitcast_convert_type(r1, jnp.uint32) & mask, jnp.float32)
    c3f = r1 - c2f
    c1 = c1f.astype(jnp.bfloat16)
    c2 = c2f.astype(jnp.bfloat16)
    c3 = c3f.astype(jnp.bfloat16)
    cb12 = jnp.concatenate([c1, c2], axis=-1)
    cbt = jnp.swapaxes(codebooks, 1, 2)

    x_recon, idx_out, loss_out = pl.pallas_call(
        _rqvae_block,
        grid=(grid,),
        in_specs=[
            pl.BlockSpec((_BLK, _IN_DIM), lambda i: (i, 0)),
            _full(ew1t.shape), _full(eb1r.shape),
            _full(ew2t.shape), _full(eb2r.shape),
            _full(ew3t.shape), _full(eb3r.shape),
            _full(dw1t.shape), _full(db1r.shape),
            _full(dw2t.shape), _full(db2r.shape),
            _full(dw3t.shape), _full(db3r.shape),
            _full(codebooks.shape),
            _full(cbt.shape),
            _full(cb12.shape),
            _full(c3.shape),
        ],
        out_specs=[
            pl.BlockSpec((_BLK, _IN_DIM), lambda i: (i, 0)),
            pl.BlockSpec((_NUM_Q, _BLK), lambda i: (0, i)),
            pl.BlockSpec((_NUM_Q, 128), lambda i: (0, 0)),
        ],
        out_shape=[
            jax.ShapeDtypeStruct((_B, _IN_DIM), jnp.float32),
            jax.ShapeDtypeStruct((_NUM_Q, _B), jnp.int32),
            jax.ShapeDtypeStruct((_NUM_Q, 128), jnp.float32),
        ],
    )(x, ew1t, eb1r, ew2t, eb2r, ew3t, eb3r,
      dw1t, db1r, dw2t, db2r, dw3t, db3r, codebooks, cbt, cb12, c3)

    indices = idx_out.T
    commitment_loss = loss_out[:, 0] / (_B * _LATENT)
    return x_recon, indices, commitment_loss
